# baseline (speedup 1.0000x reference)
import functools

import jax
import jax.numpy as jnp
from jax.experimental import pallas as pl
from jax.experimental.pallas import tpu as pltpu


def _fused_kernel(x_ref, w_ref, b_ref, g_ref, out_ref, h_ref):
    @pl.when(pl.program_id(0) == 0)
    def _():
        h_ref[:] = (
            jnp.dot(x_ref[:], w_ref[:], preferred_element_type=jnp.float32)
            + b_ref[:]
        )

    out_ref[:] = jnp.dot(g_ref[:], h_ref[:], preferred_element_type=jnp.float32)


@functools.partial(jax.jit, static_argnames=("block_m",))
def _conv_layer(x, G, W, b, block_m=400):
    n, d_in = x.shape
    d_out = W.shape[1]

    grid = (pl.cdiv(n, block_m),)
    out = pl.pallas_call(
        _fused_kernel,
        grid=grid,
        in_specs=[
            pl.BlockSpec((n, d_in), lambda i: (0, 0)),
            pl.BlockSpec((d_in, d_out), lambda i: (0, 0)),
            pl.BlockSpec((1, d_out), lambda i: (0, 0)),
            pl.BlockSpec((block_m, n), lambda i: (i, 0)),
        ],
        out_specs=pl.BlockSpec((block_m, d_out), lambda i: (i, 0)),
        out_shape=jax.ShapeDtypeStruct((n, d_out), jnp.float32),
        scratch_shapes=[pltpu.VMEM((n, d_out), jnp.float32)],
        compiler_params=pltpu.CompilerParams(
            dimension_semantics=("arbitrary",),
        ),
    )(x, W, b.reshape(1, d_out), G)
    return out


def kernel(x, G, W, b):
    out = _conv_layer(x, G, W, b)
    recon = jnp.array(0, dtype=jnp.int32)
    return (out, recon)


# speedup vs baseline: 1.0070x; 1.0070x over previous
import functools

import jax
import jax.numpy as jnp
from jax.experimental import pallas as pl
from jax.experimental.pallas import tpu as pltpu


def _fused_kernel(x_ref, w_ref, b_ref, g_ref, out_ref, h_ref):
    @pl.when(pl.program_id(0) == 0)
    def _():
        h32 = (
            jnp.dot(x_ref[:], w_ref[:], preferred_element_type=jnp.float32)
            + b_ref[:]
        )
        h_ref[:] = h32.astype(jnp.bfloat16)

    out_ref[:] = jnp.dot(
        g_ref[:].astype(jnp.bfloat16),
        h_ref[:],
        preferred_element_type=jnp.float32,
    )


@functools.partial(jax.jit, static_argnames=("block_m",))
def _conv_layer(x, G, W, b, block_m=400):
    n, d_in = x.shape
    d_out = W.shape[1]

    grid = (pl.cdiv(n, block_m),)
    out = pl.pallas_call(
        _fused_kernel,
        grid=grid,
        in_specs=[
            pl.BlockSpec((n, d_in), lambda i: (0, 0)),
            pl.BlockSpec((d_in, d_out), lambda i: (0, 0)),
            pl.BlockSpec((1, d_out), lambda i: (0, 0)),
            pl.BlockSpec((block_m, n), lambda i: (i, 0)),
        ],
        out_specs=pl.BlockSpec((block_m, d_out), lambda i: (i, 0)),
        out_shape=jax.ShapeDtypeStruct((n, d_out), jnp.float32),
        scratch_shapes=[pltpu.VMEM((n, d_out), jnp.bfloat16)],
        compiler_params=pltpu.CompilerParams(
            dimension_semantics=("arbitrary",),
        ),
    )(x, W, b.reshape(1, d_out), G)
    return out


def kernel(x, G, W, b):
    out = _conv_layer(x, G, W, b)
    recon = jnp.array(0, dtype=jnp.int32)
    return (out, recon)
